# baseline (speedup 1.0000x reference)
import functools

import numpy as np
import jax
import jax.numpy as jnp
from jax.experimental import pallas as pl
from jax.experimental.pallas import tpu as pltpu

EPS = 1e-5
_PAD = 64
_LANE = 128


def _tree_sum(vals):
    while len(vals) > 1:
        nxt = [a + b for a, b in zip(vals[::2], vals[1::2])]
        if len(vals) % 2:
            nxt.append(vals[-1])
        vals = nxt
    return vals[0]


def _fused_kernel(w_ref, x_ref, m_ref, g_ref, b_ref, o_ref,
                  s_buf, pa_buf, pb_buf, slab, asum_ref, assq_ref,
                  sc_ref, sh_ref, *, inv_m, h, w, cin, n_half):
    hw = h * w
    phase = pl.program_id(0)
    j = pl.program_id(1)

    def _build(img_slot, dst):
        s_buf[:, _PAD:_PAD + hw] = (
            x_ref[img_slot].astype(jnp.bfloat16).reshape(cin, hw))
        for dy in range(3):
            for dx in range(3):
                t = dy * 3 + dx
                s = (dy - 1) * w + (dx - 1)
                tap = s_buf[:, _PAD + s:_PAD + s + hw]
                if dx == 0:
                    tap = tap * m_ref[0:1, :]
                elif dx == 2:
                    tap = tap * m_ref[1:2, :]
                dst[t * cin:(t + 1) * cin, :] = tap

    def _dot_stats(src, slab_idx):
        y = jnp.dot(w_ref[...], src[...], preferred_element_type=jnp.float32)
        slab[slab_idx] = y.astype(slab.dtype)
        nfull = hw // _LANE
        rem = hw - nfull * _LANE
        if nfull:
            chunks = [y[:, c * _LANE:(c + 1) * _LANE] for c in range(nfull)]
            asum_ref[...] += _tree_sum(chunks)
            assq_ref[...] += _tree_sum([c * c for c in chunks])
        if rem:
            tail = y[:, nfull * _LANE:]
            asum_ref[:, :rem] += tail
            assq_ref[:, :rem] += tail * tail

    @pl.when(jnp.logical_and(phase == 0, j == 0))
    def _init():
        asum_ref[...] = jnp.zeros_like(asum_ref)
        assq_ref[...] = jnp.zeros_like(assq_ref)
        s_buf[:, :_PAD] = jnp.zeros((cin, _PAD), jnp.bfloat16)
        s_buf[:, _PAD + hw:] = jnp.zeros(
            (cin, s_buf.shape[1] - _PAD - hw), jnp.bfloat16)

    @pl.when(jnp.logical_and(phase == 0, j == 0))
    def _prologue():
        _build(0, pa_buf)
        _dot_stats(pa_buf, 0)
        _build(1, pb_buf)

    @pl.when(jnp.logical_and(phase == 0,
                             jnp.logical_and(j >= 1, j < n_half)))
    def _steady():
        _dot_stats(pb_buf, 2 * j - 1)
        _build(0, pa_buf)
        _dot_stats(pa_buf, 2 * j)
        _build(1, pb_buf)

    @pl.when(jnp.logical_and(phase == 0, j == n_half))
    def _epilogue():
        _dot_stats(pb_buf, 2 * j - 1)

    @pl.when(jnp.logical_and(phase == 1, j == 0))
    def _finalize_stats():
        mean = jnp.sum(asum_ref[...], axis=-1, keepdims=True) * inv_m
        ssq = jnp.sum(assq_ref[...], axis=-1, keepdims=True)
        var = ssq * inv_m - mean * mean
        sc = g_ref[...] * jax.lax.rsqrt(var + EPS)
        sc_ref[...] = sc
        sh_ref[...] = b_ref[...] - mean * sc

    @pl.when(jnp.logical_and(phase == 1, j < n_half))
    def _bn_relu():
        for slot in range(2):
            y = (slab[2 * j + slot].astype(jnp.float32) * sc_ref[...]
                 + sh_ref[...])
            o_ref[slot] = jnp.maximum(y, 0.0).astype(o_ref.dtype).reshape(
                o_ref.shape[1:])


def kernel(x_nchw, w_oihw, bias, gamma, beta):
    del bias
    n, cin, h, w = x_nchw.shape
    cout = w_oihw.shape[0]
    hw = h * w
    k = 9 * cin

    w_t = jnp.transpose(w_oihw, (0, 2, 3, 1)).reshape(cout, k)
    w_t = w_t.astype(jnp.bfloat16)
    g2 = gamma.reshape(cout, 1).astype(jnp.float32)
    b2 = beta.reshape(cout, 1).astype(jnp.float32)

    wpos = np.arange(hw) % w
    masks = jnp.asarray(
        np.stack([wpos >= 1, wpos <= w - 2]).astype(np.float32)
    ).astype(jnp.bfloat16)

    span = _PAD + hw + _PAD
    n_half = n // 2
    last = n_half - 1
    out = pl.pallas_call(
        functools.partial(_fused_kernel, inv_m=1.0 / (n * hw),
                          h=h, w=w, cin=cin, n_half=n_half),
        grid=(2, n_half + 1),
        in_specs=[
            pl.BlockSpec((cout, k), lambda p, j: (0, 0)),
            pl.BlockSpec((2, cin, h, w),
                         lambda p, j: ((1 - p) * jnp.minimum(j, last), 0, 0, 0)),
            pl.BlockSpec((2, hw), lambda p, j: (0, 0)),
            pl.BlockSpec((cout, 1), lambda p, j: (0, 0)),
            pl.BlockSpec((cout, 1), lambda p, j: (0, 0)),
        ],
        out_specs=pl.BlockSpec((2, cout, h, w),
                               lambda p, j: (p * jnp.minimum(j, last), 0, 0, 0)),
        out_shape=jax.ShapeDtypeStruct((n, cout, h, w), x_nchw.dtype),
        scratch_shapes=[
            pltpu.VMEM((cin, span), jnp.bfloat16),
            pltpu.VMEM((k, hw), jnp.bfloat16),
            pltpu.VMEM((k, hw), jnp.bfloat16),
            pltpu.VMEM((n, cout, hw), jnp.bfloat16),
            pltpu.VMEM((cout, _LANE), jnp.float32),
            pltpu.VMEM((cout, _LANE), jnp.float32),
            pltpu.VMEM((cout, 1), jnp.float32),
            pltpu.VMEM((cout, 1), jnp.float32),
        ],
        compiler_params=pltpu.CompilerParams(
            dimension_semantics=("arbitrary", "arbitrary"),
            vmem_limit_bytes=48 * 1024 * 1024,
        ),
    )(w_t, x_nchw, masks, g2, b2)
    return out


# speedup vs baseline: 1.3560x; 1.3560x over previous
import functools

import numpy as np
import jax
import jax.numpy as jnp
from jax.experimental import pallas as pl
from jax.experimental.pallas import tpu as pltpu

EPS = 1e-5
_PAD = 64
_LANE = 128


def _tree_sum(vals):
    while len(vals) > 1:
        nxt = [a + b for a, b in zip(vals[::2], vals[1::2])]
        if len(vals) % 2:
            nxt.append(vals[-1])
        vals = nxt
    return vals[0]


def _fused_kernel(w_ref, x_ref, m_ref, g_ref, b_ref, o_ref,
                  s_buf, pa_buf, pb_buf, slab, asum_ref, assq_ref,
                  sc_ref, sh_ref, *, inv_m, h, w, cin, n_half):
    hw = h * w
    phase = pl.program_id(0)
    j = pl.program_id(1)

    def _build(img_slot, dst):
        s_buf[:, _PAD:_PAD + hw] = x_ref[img_slot]
        for dy in range(3):
            for dx in range(3):
                t = dy * 3 + dx
                s = (dy - 1) * w + (dx - 1)
                tap = s_buf[:, _PAD + s:_PAD + s + hw]
                if dx == 0:
                    tap = tap * m_ref[0:1, :]
                elif dx == 2:
                    tap = tap * m_ref[1:2, :]
                dst[t * cin:(t + 1) * cin, :] = tap

    def _dot_stats(src, slab_idx):
        y = jnp.dot(w_ref[...], src[...], preferred_element_type=jnp.float32)
        slab[slab_idx] = y.astype(slab.dtype)
        nfull = hw // _LANE
        rem = hw - nfull * _LANE
        if nfull:
            chunks = [y[:, c * _LANE:(c + 1) * _LANE] for c in range(nfull)]
            asum_ref[...] += _tree_sum(chunks)
            assq_ref[...] += _tree_sum([c * c for c in chunks])
        if rem:
            tail = y[:, nfull * _LANE:]
            asum_ref[:, :rem] += tail
            assq_ref[:, :rem] += tail * tail

    @pl.when(jnp.logical_and(phase == 0, j == 0))
    def _init():
        asum_ref[...] = jnp.zeros_like(asum_ref)
        assq_ref[...] = jnp.zeros_like(assq_ref)
        s_buf[:, :_PAD] = jnp.zeros((cin, _PAD), jnp.bfloat16)
        s_buf[:, _PAD + hw:] = jnp.zeros(
            (cin, s_buf.shape[1] - _PAD - hw), jnp.bfloat16)

    @pl.when(jnp.logical_and(phase == 0, j == 0))
    def _prologue():
        _build(0, pa_buf)
        _dot_stats(pa_buf, 0)
        _build(1, pb_buf)

    @pl.when(jnp.logical_and(phase == 0,
                             jnp.logical_and(j >= 1, j < n_half)))
    def _steady():
        _dot_stats(pb_buf, 2 * j - 1)
        _build(0, pa_buf)
        _dot_stats(pa_buf, 2 * j)
        _build(1, pb_buf)

    @pl.when(jnp.logical_and(phase == 0, j == n_half))
    def _epilogue():
        _dot_stats(pb_buf, 2 * j - 1)

    @pl.when(jnp.logical_and(phase == 1, j == 0))
    def _finalize_stats():
        mean = jnp.sum(asum_ref[...], axis=-1, keepdims=True) * inv_m
        ssq = jnp.sum(assq_ref[...], axis=-1, keepdims=True)
        var = ssq * inv_m - mean * mean
        sc = g_ref[...] * jax.lax.rsqrt(var + EPS)
        sc_ref[...] = sc
        sh_ref[...] = b_ref[...] - mean * sc

    @pl.when(jnp.logical_and(phase == 1, j < n_half))
    def _bn_relu():
        for slot in range(2):
            y = (slab[2 * j + slot].astype(jnp.float32) * sc_ref[...]
                 + sh_ref[...])
            o_ref[slot] = jnp.maximum(y, 0.0).astype(o_ref.dtype)


def kernel(x_nchw, w_oihw, bias, gamma, beta):
    del bias
    n, cin, h, w = x_nchw.shape
    cout = w_oihw.shape[0]
    hw = h * w
    k = 9 * cin

    x_flat = x_nchw.reshape(n, cin, hw).astype(jnp.bfloat16)
    w_t = jnp.transpose(w_oihw, (0, 2, 3, 1)).reshape(cout, k)
    w_t = w_t.astype(jnp.bfloat16)
    g2 = gamma.reshape(cout, 1).astype(jnp.float32)
    b2 = beta.reshape(cout, 1).astype(jnp.float32)

    wpos = np.arange(hw) % w
    masks = jnp.asarray(
        np.stack([wpos >= 1, wpos <= w - 2]).astype(np.float32)
    ).astype(jnp.bfloat16)

    span = _PAD + hw + _PAD
    n_half = n // 2
    last = n_half - 1
    out = pl.pallas_call(
        functools.partial(_fused_kernel, inv_m=1.0 / (n * hw),
                          h=h, w=w, cin=cin, n_half=n_half),
        grid=(2, n_half + 1),
        in_specs=[
            pl.BlockSpec((cout, k), lambda p, j: (0, 0)),
            pl.BlockSpec((2, cin, hw),
                         lambda p, j: ((1 - p) * jnp.minimum(j, last), 0, 0)),
            pl.BlockSpec((2, hw), lambda p, j: (0, 0)),
            pl.BlockSpec((cout, 1), lambda p, j: (0, 0)),
            pl.BlockSpec((cout, 1), lambda p, j: (0, 0)),
        ],
        out_specs=pl.BlockSpec((2, cout, hw),
                               lambda p, j: (p * jnp.minimum(j, last), 0, 0)),
        out_shape=jax.ShapeDtypeStruct((n, cout, hw), jnp.bfloat16),
        scratch_shapes=[
            pltpu.VMEM((cin, span), jnp.bfloat16),
            pltpu.VMEM((k, hw), jnp.bfloat16),
            pltpu.VMEM((k, hw), jnp.bfloat16),
            pltpu.VMEM((n, cout, hw), jnp.bfloat16),
            pltpu.VMEM((cout, _LANE), jnp.float32),
            pltpu.VMEM((cout, _LANE), jnp.float32),
            pltpu.VMEM((cout, 1), jnp.float32),
            pltpu.VMEM((cout, 1), jnp.float32),
        ],
        compiler_params=pltpu.CompilerParams(
            dimension_semantics=("arbitrary", "arbitrary"),
            vmem_limit_bytes=48 * 1024 * 1024,
        ),
    )(w_t, x_flat, masks, g2, b2)
    return out.astype(x_nchw.dtype).reshape(n, cout, h, w)


# speedup vs baseline: 1.4580x; 1.0752x over previous
import functools

import numpy as np
import jax
import jax.numpy as jnp
from jax.experimental import pallas as pl
from jax.experimental.pallas import tpu as pltpu

EPS = 1e-5
_PAD = 64
_LANE = 128


def _tree_sum(vals):
    while len(vals) > 1:
        nxt = [a + b for a, b in zip(vals[::2], vals[1::2])]
        if len(vals) % 2:
            nxt.append(vals[-1])
        vals = nxt
    return vals[0]


def _fused_kernel(w_ref, x_ref, m_ref, g_ref, b_ref, o_ref,
                  s_c, s_l, s_r, pa_buf, pb_buf, slab, asum_ref, assq_ref,
                  sc_ref, sh_ref, *, inv_m, h, w, cin, n_half):
    hw = h * w
    phase = pl.program_id(0)
    j = pl.program_id(1)

    def _build(img_slot, dst):
        xc = x_ref[img_slot].astype(jnp.bfloat16)
        s_c[:, _PAD:_PAD + hw] = xc
        s_l[:, _PAD:_PAD + hw] = xc * m_ref[0:1, :]
        s_r[:, _PAD:_PAD + hw] = xc * m_ref[1:2, :]
        srcs = (s_l, s_c, s_r)
        for dy in range(3):
            for dx in range(3):
                t = dy * 3 + dx
                s = (dy - 1) * w + (dx - 1)
                src = srcs[dx]
                dst[t * cin:(t + 1) * cin, :] = src[:, _PAD + s:_PAD + s + hw]

    def _dot_stats(src, slab_idx):
        y = jnp.dot(w_ref[...], src[...], preferred_element_type=jnp.float32)
        slab[slab_idx] = y.astype(slab.dtype)
        nfull = hw // _LANE
        rem = hw - nfull * _LANE
        if nfull:
            chunks = [y[:, c * _LANE:(c + 1) * _LANE] for c in range(nfull)]
            asum_ref[...] += _tree_sum(chunks)
            assq_ref[...] += _tree_sum([c * c for c in chunks])
        if rem:
            tail = y[:, nfull * _LANE:]
            asum_ref[:, :rem] += tail
            assq_ref[:, :rem] += tail * tail

    @pl.when(jnp.logical_and(phase == 0, j == 0))
    def _init():
        asum_ref[...] = jnp.zeros_like(asum_ref)
        assq_ref[...] = jnp.zeros_like(assq_ref)
        for sb in (s_c, s_l, s_r):
            sb[:, :_PAD] = jnp.zeros((cin, _PAD), jnp.bfloat16)
            sb[:, _PAD + hw:] = jnp.zeros(
                (cin, sb.shape[1] - _PAD - hw), jnp.bfloat16)

    @pl.when(jnp.logical_and(phase == 0, j == 0))
    def _prologue():
        _build(0, pa_buf)
        _dot_stats(pa_buf, 0)
        _build(1, pb_buf)

    @pl.when(jnp.logical_and(phase == 0,
                             jnp.logical_and(j >= 1, j < n_half)))
    def _steady():
        _dot_stats(pb_buf, 2 * j - 1)
        _build(0, pa_buf)
        _dot_stats(pa_buf, 2 * j)
        _build(1, pb_buf)

    @pl.when(jnp.logical_and(phase == 0, j == n_half))
    def _epilogue():
        _dot_stats(pb_buf, 2 * j - 1)

    @pl.when(jnp.logical_and(phase == 1, j == 0))
    def _finalize_stats():
        mean = jnp.sum(asum_ref[...], axis=-1, keepdims=True) * inv_m
        ssq = jnp.sum(assq_ref[...], axis=-1, keepdims=True)
        var = ssq * inv_m - mean * mean
        sc = g_ref[...] * jax.lax.rsqrt(var + EPS)
        sc_ref[...] = sc
        sh_ref[...] = b_ref[...] - mean * sc

    @pl.when(jnp.logical_and(phase == 1, j < n_half))
    def _bn_relu():
        for slot in range(2):
            y = (slab[2 * j + slot].astype(jnp.float32) * sc_ref[...]
                 + sh_ref[...])
            o_ref[slot] = jnp.maximum(y, 0.0).astype(o_ref.dtype)


def kernel(x_nchw, w_oihw, bias, gamma, beta):
    del bias
    n, cin, h, w = x_nchw.shape
    cout = w_oihw.shape[0]
    hw = h * w
    k = 9 * cin

    x_flat = x_nchw.reshape(n, cin, hw)
    w_t = jnp.transpose(w_oihw, (0, 2, 3, 1)).reshape(cout, k)
    w_t = w_t.astype(jnp.bfloat16)
    g2 = gamma.reshape(cout, 1).astype(jnp.float32)
    b2 = beta.reshape(cout, 1).astype(jnp.float32)

    wpos = np.arange(hw) % w
    masks = jnp.asarray(
        np.stack([wpos != w - 1, wpos != 0]).astype(np.float32)
    ).astype(jnp.bfloat16)

    span = _PAD + hw + _PAD
    n_half = n // 2
    last = n_half - 1
    out = pl.pallas_call(
        functools.partial(_fused_kernel, inv_m=1.0 / (n * hw),
                          h=h, w=w, cin=cin, n_half=n_half),
        grid=(2, n_half + 1),
        in_specs=[
            pl.BlockSpec((cout, k), lambda p, j: (0, 0)),
            pl.BlockSpec((2, cin, hw),
                         lambda p, j: ((1 - p) * jnp.minimum(j, last), 0, 0)),
            pl.BlockSpec((2, hw), lambda p, j: (0, 0)),
            pl.BlockSpec((cout, 1), lambda p, j: (0, 0)),
            pl.BlockSpec((cout, 1), lambda p, j: (0, 0)),
        ],
        out_specs=pl.BlockSpec((2, cout, hw),
                               lambda p, j: (p * jnp.minimum(j, last), 0, 0)),
        out_shape=jax.ShapeDtypeStruct((n, cout, hw), x_nchw.dtype),
        scratch_shapes=[
            pltpu.VMEM((cin, span), jnp.bfloat16),
            pltpu.VMEM((cin, span), jnp.bfloat16),
            pltpu.VMEM((cin, span), jnp.bfloat16),
            pltpu.VMEM((k, hw), jnp.bfloat16),
            pltpu.VMEM((k, hw), jnp.bfloat16),
            pltpu.VMEM((n, cout, hw), jnp.bfloat16),
            pltpu.VMEM((cout, _LANE), jnp.float32),
            pltpu.VMEM((cout, _LANE), jnp.float32),
            pltpu.VMEM((cout, 1), jnp.float32),
            pltpu.VMEM((cout, 1), jnp.float32),
        ],
        compiler_params=pltpu.CompilerParams(
            dimension_semantics=("arbitrary", "arbitrary"),
            vmem_limit_bytes=48 * 1024 * 1024,
        ),
    )(w_t, x_flat, masks, g2, b2)
    return out.reshape(n, cout, h, w)


# speedup vs baseline: 1.6224x; 1.1128x over previous
import functools

import numpy as np
import jax
import jax.numpy as jnp
from jax.experimental import pallas as pl
from jax.experimental.pallas import tpu as pltpu

EPS = 1e-5
_PAD = 128
_LANE = 128


def _tree_sum(vals):
    while len(vals) > 1:
        nxt = [a + b for a, b in zip(vals[::2], vals[1::2])]
        if len(vals) % 2:
            nxt.append(vals[-1])
        vals = nxt
    return vals[0]


def _fused_kernel(w_ref, x_ref, m_ref, g_ref, b_ref, o_ref,
                  s_c, s_l, s_r, pa_buf, pb_buf, slab, asum_ref, assq_ref,
                  sc_ref, sh_ref, *, inv_m, h, w, cin, n_half):
    hw = h * w
    phase = pl.program_id(0)
    j = pl.program_id(1)

    def _build(img_slot, dst):
        xc = x_ref[img_slot].astype(jnp.bfloat16)
        s_c[:, _PAD:_PAD + hw] = xc
        s_l[:, _PAD:_PAD + hw] = xc * m_ref[0:1, :]
        s_r[:, _PAD:_PAD + hw] = xc * m_ref[1:2, :]
        srcs = (s_l, s_c, s_r)
        for dy in range(3):
            for dx in range(3):
                t = dy * 3 + dx
                s = (dy - 1) * w + (dx - 1)
                src = srcs[dx]
                dst[t * cin:(t + 1) * cin, :] = src[:, _PAD + s:_PAD + s + hw]

    def _dot_stats(src, slab_idx):
        y = jnp.dot(w_ref[...], src[...], preferred_element_type=jnp.float32)
        slab[slab_idx] = y.astype(slab.dtype)
        nfull = hw // _LANE
        rem = hw - nfull * _LANE
        if nfull:
            chunks = [y[:, c * _LANE:(c + 1) * _LANE] for c in range(nfull)]
            asum_ref[...] += _tree_sum(chunks)
            assq_ref[...] += _tree_sum([c * c for c in chunks])
        if rem:
            tail = y[:, nfull * _LANE:]
            asum_ref[:, :rem] += tail
            assq_ref[:, :rem] += tail * tail

    @pl.when(jnp.logical_and(phase == 0, j == 0))
    def _init():
        asum_ref[...] = jnp.zeros_like(asum_ref)
        assq_ref[...] = jnp.zeros_like(assq_ref)
        for sb in (s_c, s_l, s_r):
            sb[:, :_PAD] = jnp.zeros((cin, _PAD), jnp.bfloat16)
            sb[:, _PAD + hw:] = jnp.zeros(
                (cin, sb.shape[1] - _PAD - hw), jnp.bfloat16)

    @pl.when(jnp.logical_and(phase == 0, j == 0))
    def _prologue():
        _build(0, pa_buf)
        _dot_stats(pa_buf, 0)
        _build(1, pb_buf)

    @pl.when(jnp.logical_and(phase == 0,
                             jnp.logical_and(j >= 1, j < n_half)))
    def _steady():
        _dot_stats(pb_buf, 2 * j - 1)
        _build(0, pa_buf)
        _dot_stats(pa_buf, 2 * j)
        _build(1, pb_buf)

    @pl.when(jnp.logical_and(phase == 0, j == n_half))
    def _epilogue():
        _dot_stats(pb_buf, 2 * j - 1)

    @pl.when(jnp.logical_and(phase == 1, j == 0))
    def _finalize_stats():
        mean = jnp.sum(asum_ref[...], axis=-1, keepdims=True) * inv_m
        ssq = jnp.sum(assq_ref[...], axis=-1, keepdims=True)
        var = ssq * inv_m - mean * mean
        sc = g_ref[...] * jax.lax.rsqrt(var + EPS)
        sc_ref[...] = sc
        sh_ref[...] = b_ref[...] - mean * sc

    @pl.when(jnp.logical_and(phase == 1, j < n_half))
    def _bn_relu():
        for slot in range(2):
            y = (slab[2 * j + slot].astype(jnp.float32) * sc_ref[...]
                 + sh_ref[...])
            o_ref[slot] = jnp.maximum(y, 0.0).astype(o_ref.dtype)


def kernel(x_nchw, w_oihw, bias, gamma, beta):
    del bias
    n, cin, h, w = x_nchw.shape
    cout = w_oihw.shape[0]
    hw = h * w
    k = 9 * cin

    x_flat = x_nchw.reshape(n, cin, hw)
    w_t = jnp.transpose(w_oihw, (0, 2, 3, 1)).reshape(cout, k)
    w_t = w_t.astype(jnp.bfloat16)
    g2 = gamma.reshape(cout, 1).astype(jnp.float32)
    b2 = beta.reshape(cout, 1).astype(jnp.float32)

    wpos = np.arange(hw) % w
    masks = jnp.asarray(
        np.stack([wpos != w - 1, wpos != 0]).astype(np.float32)
    ).astype(jnp.bfloat16)

    span = _PAD + hw + _PAD
    n_half = n // 2
    last = n_half - 1
    out = pl.pallas_call(
        functools.partial(_fused_kernel, inv_m=1.0 / (n * hw),
                          h=h, w=w, cin=cin, n_half=n_half),
        grid=(2, n_half + 1),
        in_specs=[
            pl.BlockSpec((cout, k), lambda p, j: (0, 0)),
            pl.BlockSpec((2, cin, hw),
                         lambda p, j: ((1 - p) * jnp.minimum(j, last), 0, 0)),
            pl.BlockSpec((2, hw), lambda p, j: (0, 0)),
            pl.BlockSpec((cout, 1), lambda p, j: (0, 0)),
            pl.BlockSpec((cout, 1), lambda p, j: (0, 0)),
        ],
        out_specs=pl.BlockSpec((2, cout, hw),
                               lambda p, j: (p * jnp.minimum(j, last), 0, 0)),
        out_shape=jax.ShapeDtypeStruct((n, cout, hw), x_nchw.dtype),
        scratch_shapes=[
            pltpu.VMEM((cin, span), jnp.bfloat16),
            pltpu.VMEM((cin, span), jnp.bfloat16),
            pltpu.VMEM((cin, span), jnp.bfloat16),
            pltpu.VMEM((k, hw), jnp.bfloat16),
            pltpu.VMEM((k, hw), jnp.bfloat16),
            pltpu.VMEM((n, cout, hw), jnp.bfloat16),
            pltpu.VMEM((cout, _LANE), jnp.float32),
            pltpu.VMEM((cout, _LANE), jnp.float32),
            pltpu.VMEM((cout, 1), jnp.float32),
            pltpu.VMEM((cout, 1), jnp.float32),
        ],
        compiler_params=pltpu.CompilerParams(
            dimension_semantics=("arbitrary", "arbitrary"),
            vmem_limit_bytes=48 * 1024 * 1024,
        ),
    )(w_t, x_flat, masks, g2, b2)
    return out.reshape(n, cout, h, w)
